# Initial kernel scaffold; baseline (speedup 1.0000x reference)
import functools

import jax
import jax.numpy as jnp
from jax import lax
from jax.experimental import pallas as pl
from jax.experimental.pallas import tpu as pltpu
from jax.experimental.pallas import tpu_sc as plsc

NC = 2
NS = 16
NW = NC * NS
GB = 128

_STEPS = 4


def _mesh():
    return plsc.VectorSubcoreMesh(
        core_axis_name="c", subcore_axis_name="s", num_cores=NC, num_subcores=NS
    )


@functools.lru_cache(maxsize=None)
def _make_gather(N, E, D):
    assert E % NW == 0
    epw = E // NW
    nb = epw // GB
    tail = epw - nb * GB
    assert epw % 8 == 0 and tail % 8 == 0

    def body(h_hbm, src_hbm, out_hbm, idx_b, rows, idx_t, rows_t, sem):
        c = lax.axis_index("c")
        s = lax.axis_index("s")
        base = (s * NC + c) * epw

        def step(j, carry):
            off = pl.multiple_of(base + j * GB, 8)
            pltpu.sync_copy(src_hbm.at[pl.ds(off, GB)], idx_b)
            pltpu.async_copy(h_hbm.at[idx_b], rows, sem).wait()
            pltpu.sync_copy(rows, out_hbm.at[pl.ds(off, GB)])
            return carry

        lax.fori_loop(0, nb, step, 0)
        if tail:
            off = pl.multiple_of(base + nb * GB, 8)
            pltpu.sync_copy(src_hbm.at[pl.ds(off, tail)], idx_t)
            pltpu.async_copy(h_hbm.at[idx_t], rows_t, sem).wait()
            pltpu.sync_copy(rows_t, out_hbm.at[pl.ds(off, tail)])

    return pl.kernel(
        body,
        out_type=jax.ShapeDtypeStruct((E, D), jnp.float32),
        mesh=_mesh(),
        scratch_types=[
            pltpu.VMEM((GB,), jnp.int32),
            pltpu.VMEM((GB, D), jnp.float32),
            pltpu.VMEM((max(tail, 8),), jnp.int32),
            pltpu.VMEM((max(tail, 8), D), jnp.float32),
            pltpu.SemaphoreType.DMA,
        ],
    )


@functools.lru_cache(maxsize=None)
def _make_segsum(N, E, D):
    hc = D // NC
    ept = E // NS
    nb = ept // GB
    tail = ept - nb * GB
    rpt = N // NS
    assert E % NS == 0 and N % NS == 0 and tail % 8 == 0 and rpt % 8 == 0
    zr = 125 if rpt % 125 == 0 else rpt
    nz = rpt // zr

    def body(t_hbm, dst_hbm, agg_hbm, buf, idx_b, buf_t, idx_t, zbuf, acc, sem):
        c = lax.axis_index("c")
        s = lax.axis_index("s")

        def zstep(i, carry):
            r = i // (hc // 16)
            q = (i % (hc // 16)) * 16
            zbuf[r, pl.ds(q, 16)] = jnp.zeros((16,), jnp.float32)
            return carry

        lax.fori_loop(0, zr * (hc // 16), zstep, 0)
        for r in range(nz):
            pltpu.sync_copy(zbuf, acc.at[pl.ds(s * rpt + r * zr, zr)])
        plsc.subcore_barrier()

        base = s * ept

        def step(j, carry):
            off = pl.multiple_of(base + j * GB, 8)
            pltpu.sync_copy(t_hbm.at[c, pl.ds(off, GB)], buf)
            pltpu.sync_copy(dst_hbm.at[pl.ds(off, GB)], idx_b)
            pltpu.sync_copy(buf, acc.at[idx_b], add=True)
            return carry

        lax.fori_loop(0, nb, step, 0)
        if tail:
            off = pl.multiple_of(base + nb * GB, 8)
            pltpu.sync_copy(t_hbm.at[c, pl.ds(off, tail)], buf_t)
            pltpu.sync_copy(dst_hbm.at[pl.ds(off, tail)], idx_t)
            pltpu.sync_copy(buf_t, acc.at[idx_t], add=True)
        plsc.subcore_barrier()

        r0 = s * rpt
        pltpu.sync_copy(
            acc.at[pl.ds(r0, rpt)], agg_hbm.at[pl.ds(r0, rpt), pl.ds(c * hc, hc)]
        )

    return pl.kernel(
        body,
        out_type=jax.ShapeDtypeStruct((N, D), jnp.float32),
        mesh=_mesh(),
        scratch_types=[
            pltpu.VMEM((GB, hc), jnp.float32),
            pltpu.VMEM((GB,), jnp.int32),
            pltpu.VMEM((max(tail, 8), hc), jnp.float32),
            pltpu.VMEM((max(tail, 8),), jnp.int32),
            pltpu.VMEM((zr, hc), jnp.float32),
            pltpu.VMEM_SHARED((N, hc), jnp.float32),
            pltpu.SemaphoreType.DMA,
        ],
    )


def _edge_transform(xj, bond, wstack):
    E, D = xj.shape
    BDp1 = wstack.shape[0]
    hc = D // NC
    be = 2000
    assert E % be == 0

    def body(xj_ref, b_ref, w_ref, o_ref):
        x = xj_ref[...]
        acc = jnp.dot(x, w_ref[0], preferred_element_type=jnp.float32)
        for k in range(1, BDp1):
            acc += b_ref[:, k - 1 : k] * jnp.dot(
                x, w_ref[k], preferred_element_type=jnp.float32
            )
        for c in range(NC):
            o_ref[c] = acc[:, c * hc : (c + 1) * hc]

    return pl.pallas_call(
        body,
        grid=(E // be,),
        in_specs=[
            pl.BlockSpec((be, D), lambda i: (i, 0)),
            pl.BlockSpec((be, bond.shape[1]), lambda i: (i, 0)),
            pl.BlockSpec((BDp1, D, D), lambda i: (0, 0, 0)),
        ],
        out_specs=pl.BlockSpec((NC, be, hc), lambda i: (0, i, 0)),
        out_shape=jax.ShapeDtypeStruct((NC, E, hc), jnp.float32),
    )(xj, bond, wstack)


def _gru(agg, h, wk, wr, b):
    N, D = h.shape
    bn = 2000
    assert N % bn == 0

    def body(a_ref, h_ref, wk_ref, wr_ref, b_ref, o_ref):
        a = a_ref[...]
        hh = h_ref[...]
        xp = jnp.dot(a, wk_ref[...], preferred_element_type=jnp.float32) + b_ref[0]
        hp = jnp.dot(hh, wr_ref[...], preferred_element_type=jnp.float32) + b_ref[1]
        z = jax.nn.sigmoid(xp[:, :D] + hp[:, :D])
        r = jax.nn.sigmoid(xp[:, D : 2 * D] + hp[:, D : 2 * D])
        cand = jnp.tanh(xp[:, 2 * D :] + r * hp[:, 2 * D :])
        o_ref[...] = z * hh + (1.0 - z) * cand

    return pl.pallas_call(
        body,
        grid=(N // bn,),
        in_specs=[
            pl.BlockSpec((bn, D), lambda i: (i, 0)),
            pl.BlockSpec((bn, D), lambda i: (i, 0)),
            pl.BlockSpec((D, 3 * D), lambda i: (0, 0)),
            pl.BlockSpec((D, 3 * D), lambda i: (0, 0)),
            pl.BlockSpec((2, 3 * D), lambda i: (0, 0)),
        ],
        out_specs=pl.BlockSpec((bn, D), lambda i: (i, 0)),
        out_shape=jax.ShapeDtypeStruct((N, D), jnp.float32),
    )(agg, h, wk, wr, b)


def kernel(atom_features, bond_features, pair_indices, kernel, bias, gru_kernel,
           gru_rec_kernel, gru_bias):
    N, D = atom_features.shape
    E, BD = bond_features.shape
    dst = pair_indices[:, 0].astype(jnp.int32)
    src = pair_indices[:, 1].astype(jnp.int32)
    Kr = kernel.reshape(BD, D, D)
    wstack = jnp.concatenate(
        [bias.reshape(D, D).T[None], jnp.transpose(Kr, (0, 2, 1))], axis=0
    )
    gather = _make_gather(N, E, D)
    segsum = _make_segsum(N, E, D)

    h = atom_features
    for _ in range(_STEPS):
        xj = gather(h, src)
        t2 = _edge_transform(xj, bond_features, wstack)
        agg = segsum(t2, dst)
        h = _gru(agg, h, gru_kernel, gru_rec_kernel, gru_bias)
    return h


# speedup vs baseline: 3.9564x; 3.9564x over previous
import functools

import jax
import jax.numpy as jnp
from jax import lax
from jax.experimental import pallas as pl
from jax.experimental.pallas import tpu as pltpu
from jax.experimental.pallas import tpu_sc as plsc

NC = 2
NS = 16
NW = NC * NS
GB = 128

_STEPS = 4


def _mesh():
    return plsc.VectorSubcoreMesh(
        core_axis_name="c", subcore_axis_name="s", num_cores=NC, num_subcores=NS
    )


@functools.lru_cache(maxsize=None)
def _make_gather(N, E, D):
    assert E % NW == 0
    epw = E // NW
    nb = epw // GB
    tail = epw - nb * GB
    assert epw % 8 == 0 and tail % 8 == 0

    def body(h_hbm, src_hbm, out_hbm, idx_b, rows, idx_t, rows_t, sem):
        c = lax.axis_index("c")
        s = lax.axis_index("s")
        base = (s * NC + c) * epw

        def step(j, carry):
            off = pl.multiple_of(base + j * GB, 8)
            pltpu.sync_copy(src_hbm.at[pl.ds(off, GB)], idx_b)
            pltpu.async_copy(h_hbm.at[idx_b], rows, sem).wait()
            pltpu.sync_copy(rows, out_hbm.at[pl.ds(off, GB)])
            return carry

        lax.fori_loop(0, nb, step, 0)
        if tail:
            off = pl.multiple_of(base + nb * GB, 8)
            pltpu.sync_copy(src_hbm.at[pl.ds(off, tail)], idx_t)
            pltpu.async_copy(h_hbm.at[idx_t], rows_t, sem).wait()
            pltpu.sync_copy(rows_t, out_hbm.at[pl.ds(off, tail)])

    return pl.kernel(
        body,
        out_type=jax.ShapeDtypeStruct((E, D), jnp.float32),
        mesh=_mesh(),
        scratch_types=[
            pltpu.VMEM((GB,), jnp.int32),
            pltpu.VMEM((GB, D), jnp.float32),
            pltpu.VMEM((max(tail, 8),), jnp.int32),
            pltpu.VMEM((max(tail, 8), D), jnp.float32),
            pltpu.SemaphoreType.DMA,
        ],
    )


@functools.lru_cache(maxsize=None)
def _make_segsum(N, E, D):
    epw = E // NW
    nb = epw // GB
    tail = epw - nb * GB
    rpt = -(-(N // NS) // 8) * 8
    rlast = N - (NS - 1) * rpt
    assert E % NW == 0 and tail % 8 == 0 and 0 < rlast <= rpt

    def body(t_hbm, dst_hbm, agg_hbm, buf, idx_b, buf_t, idx_t, zbuf, acc, sem):
        c = lax.axis_index("c")
        s = lax.axis_index("s")

        def zstep(i, carry):
            r = i // (D // 16)
            q = (i % (D // 16)) * 16
            zbuf[r, pl.ds(q, 16)] = jnp.zeros((16,), jnp.float32)
            return carry

        lax.fori_loop(0, 8 * (D // 16), zstep, 0)
        r0 = pl.multiple_of(s * rpt, 8)
        nrep = lax.select(s < NS - 1, rpt // 8, rlast // 8)

        def zcopy(r, carry):
            pltpu.sync_copy(zbuf, acc.at[pl.ds(r0 + r * 8, 8)])
            return carry

        lax.fori_loop(0, nrep, zcopy, 0)
        plsc.subcore_barrier()

        base = (c * NS + s) * epw

        def step(j, carry):
            off = pl.multiple_of(base + j * GB, 8)
            pltpu.sync_copy(t_hbm.at[pl.ds(off, GB)], buf)
            pltpu.sync_copy(dst_hbm.at[pl.ds(off, GB)], idx_b)
            pltpu.sync_copy(buf, acc.at[idx_b], add=True)
            return carry

        lax.fori_loop(0, nb, step, 0)
        if tail:
            off = pl.multiple_of(base + nb * GB, 8)
            pltpu.sync_copy(t_hbm.at[pl.ds(off, tail)], buf_t)
            pltpu.sync_copy(dst_hbm.at[pl.ds(off, tail)], idx_t)
            pltpu.sync_copy(buf_t, acc.at[idx_t], add=True)
        plsc.subcore_barrier()

        @pl.when(s < NS - 1)
        def _():
            pltpu.sync_copy(
                acc.at[pl.ds(r0, rpt)], agg_hbm.at[c, pl.ds(r0, rpt)]
            )

        @pl.when(s == NS - 1)
        def _():
            pltpu.sync_copy(
                acc.at[pl.ds(r0, rlast)], agg_hbm.at[c, pl.ds(r0, rlast)]
            )

    return pl.kernel(
        body,
        out_type=jax.ShapeDtypeStruct((NC, N, D), jnp.float32),
        mesh=_mesh(),
        scratch_types=[
            pltpu.VMEM((GB, D), jnp.float32),
            pltpu.VMEM((GB,), jnp.int32),
            pltpu.VMEM((max(tail, 8), D), jnp.float32),
            pltpu.VMEM((max(tail, 8),), jnp.int32),
            pltpu.VMEM((8, D), jnp.float32),
            pltpu.VMEM_SHARED((N, D), jnp.float32),
            pltpu.SemaphoreType.DMA,
        ],
    )


def _edge_transform(xj, bond, wstack):
    E, D = xj.shape
    BDp1 = wstack.shape[0]
    be = 2000
    assert E % be == 0

    def body(xj_ref, b_ref, w_ref, o_ref):
        x = xj_ref[...]
        acc = jnp.dot(x, w_ref[0], preferred_element_type=jnp.float32)
        for k in range(1, BDp1):
            acc += b_ref[:, k - 1 : k] * jnp.dot(
                x, w_ref[k], preferred_element_type=jnp.float32
            )
        o_ref[...] = acc

    return pl.pallas_call(
        body,
        grid=(E // be,),
        in_specs=[
            pl.BlockSpec((be, D), lambda i: (i, 0)),
            pl.BlockSpec((be, bond.shape[1]), lambda i: (i, 0)),
            pl.BlockSpec((BDp1, D, D), lambda i: (0, 0, 0)),
        ],
        out_specs=pl.BlockSpec((be, D), lambda i: (i, 0)),
        out_shape=jax.ShapeDtypeStruct((E, D), jnp.float32),
    )(xj, bond, wstack)


def _gru(agg2, h, wk, wr, b):
    N, D = h.shape
    bn = 2000
    assert N % bn == 0

    def body(a_ref, h_ref, wk_ref, wr_ref, b_ref, o_ref):
        a = a_ref[0] + a_ref[1]
        hh = h_ref[...]
        xp = jnp.dot(a, wk_ref[...], preferred_element_type=jnp.float32) + b_ref[0]
        hp = jnp.dot(hh, wr_ref[...], preferred_element_type=jnp.float32) + b_ref[1]
        z = jax.nn.sigmoid(xp[:, :D] + hp[:, :D])
        r = jax.nn.sigmoid(xp[:, D : 2 * D] + hp[:, D : 2 * D])
        cand = jnp.tanh(xp[:, 2 * D :] + r * hp[:, 2 * D :])
        o_ref[...] = z * hh + (1.0 - z) * cand

    return pl.pallas_call(
        body,
        grid=(N // bn,),
        in_specs=[
            pl.BlockSpec((NC, bn, D), lambda i: (0, i, 0)),
            pl.BlockSpec((bn, D), lambda i: (i, 0)),
            pl.BlockSpec((D, 3 * D), lambda i: (0, 0)),
            pl.BlockSpec((D, 3 * D), lambda i: (0, 0)),
            pl.BlockSpec((2, 3 * D), lambda i: (0, 0)),
        ],
        out_specs=pl.BlockSpec((bn, D), lambda i: (i, 0)),
        out_shape=jax.ShapeDtypeStruct((N, D), jnp.float32),
    )(agg2, h, wk, wr, b)


def kernel(atom_features, bond_features, pair_indices, kernel, bias, gru_kernel,
           gru_rec_kernel, gru_bias):
    N, D = atom_features.shape
    E, BD = bond_features.shape
    dst = pair_indices[:, 0].astype(jnp.int32)
    src = pair_indices[:, 1].astype(jnp.int32)
    Kr = kernel.reshape(BD, D, D)
    wstack = jnp.concatenate(
        [bias.reshape(D, D).T[None], jnp.transpose(Kr, (0, 2, 1))], axis=0
    )
    gather = _make_gather(N, E, D)
    segsum = _make_segsum(N, E, D)

    h = atom_features
    for _ in range(_STEPS):
        xj = gather(h, src)
        t2 = _edge_transform(xj, bond_features, wstack)
        agg = segsum(t2, dst)
        h = _gru(agg, h, gru_kernel, gru_rec_kernel, gru_bias)
    return h
